# baseline (speedup 1.0000x reference)
import functools

import jax
import jax.numpy as jnp
from jax import lax
from jax.experimental import pallas as pl
from jax.experimental.pallas import tpu as pltpu
from jax.experimental.pallas import tpu_sc as plsc

B = 16384
N_CONT = 13
N_BINARY = 26
N_CAT = 26
VOCAB = 100000
D_F = 32
N_FEAT = N_CONT + N_BINARY + N_CAT

NC, NS = 2, 16
NW = NC * NS
TOTAL_IDX = B * N_CAT
IDX_PER_W = TOTAL_IDX // NW
CHUNK = 1664

BB = 1024
G_ROWS_PER_FIELD = B * D_F // 128
G_BLK = BB * D_F // 128

FSPLITS = ((0, 7), (7, 14), (14, 20), (20, 26))


def _mk_gather_body(n_idx, n_per_w, chunk):
    def _gather_body(table_hbm, idx_hbm, out_hbm, idx_v, rows_v, sem):
        wid = lax.axis_index("s") * NC + lax.axis_index("c")
        base = wid * n_per_w

        @pl.loop(0, n_per_w, step=chunk)
        def _(off):
            pltpu.sync_copy(idx_hbm.at[pl.ds(base + off, chunk)], idx_v)
            pltpu.async_copy(table_hbm.at[idx_v], rows_v, sem).wait()
            pltpu.sync_copy(rows_v, out_hbm.at[pl.ds(base + off, chunk)])

    return _gather_body


def _cat_gather(table_flat, idx_flat, nf):
    n_idx = B * nf
    n_per_w = n_idx // NW
    chunk = n_per_w // 4
    mesh = plsc.VectorSubcoreMesh(core_axis_name="c", subcore_axis_name="s")
    k = pl.kernel(
        _mk_gather_body(n_idx, n_per_w, chunk),
        out_type=jax.ShapeDtypeStruct((n_idx, D_F), jnp.float32),
        mesh=mesh,
        scratch_types=[
            pltpu.VMEM((chunk,), jnp.int32),
            pltpu.VMEM((chunk, D_F), jnp.float32),
            pltpu.SemaphoreType.DMA,
        ],
        compiler_params=pltpu.CompilerParams(use_tc_tiling_on_sc=False),
    )
    return k(table_flat, idx_flat)


def _asm_body(xc_ref, xb_ref, wb_ref, t0_ref, dt_ref, *rest):
    g_refs, o_ref = rest[:N_CAT], rest[N_CAT]
    wcol = wb_ref[:, 0:1]
    bcol = wb_ref[:, 1:2]
    for i in range(N_CONT):
        o_ref[i] = wcol * xc_ref[i:i + 1, :] + bcol
    for i in range(N_BINARY):
        o_ref[N_CONT + i] = (
            t0_ref[:, i:i + 1] + dt_ref[:, i:i + 1] * xb_ref[i:i + 1, :]
        )
    for i in range(N_CAT):
        o_ref[N_CONT + N_BINARY + i] = jnp.transpose(g_refs[i][0])


def _assemble(xc_t, xb_t, wb, t0_t, dt_t, g3s):
    nb = B // BB

    def _mk(li):
        return pl.BlockSpec((1, BB, D_F), lambda j, li=li: (li, j, 0))

    in_specs = [
        pl.BlockSpec((N_CONT, BB), lambda j: (0, j)),
        pl.BlockSpec((N_BINARY, BB), lambda j: (0, j)),
        pl.BlockSpec((D_F, 2), lambda j: (0, 0)),
        pl.BlockSpec((D_F, N_BINARY), lambda j: (0, 0)),
        pl.BlockSpec((D_F, N_BINARY), lambda j: (0, 0)),
    ]
    g_args = []
    for k, (a, b) in enumerate(FSPLITS):
        for i in range(a, b):
            in_specs.append(_mk(i - a))
            g_args.append(g3s[k])

    return pl.pallas_call(
        _asm_body,
        grid=(nb,),
        in_specs=in_specs,
        out_specs=pl.BlockSpec((N_FEAT, D_F, BB), lambda j: (0, 0, j)),
        out_shape=jax.ShapeDtypeStruct((N_FEAT, D_F, B), jnp.float32),
    )(xc_t, xb_t, wb, t0_t, dt_t, *g_args)


def kernel(x_cont, x_binary, x_cat, W_cont, b_cont, binary_tables, cat_tables):
    xct = x_cat.T.astype(jnp.int32)

    xc_t = x_cont.T
    xb_t = x_binary.T.astype(jnp.float32)
    wb = jnp.stack([W_cont[0], b_cont], axis=1)
    t0_t = binary_tables[:, 0, :].T
    dt_t = (binary_tables[:, 1, :] - binary_tables[:, 0, :]).T

    g3s = []
    for (a, b) in FSPLITS:
        nf = b - a
        idx_k = (
            xct[a:b] + (jnp.arange(nf, dtype=jnp.int32) * VOCAB)[:, None]
        ).reshape(B * nf)
        tbl_k = cat_tables[a:b].reshape(nf * VOCAB, D_F)
        g_k = _cat_gather(tbl_k, idx_k, nf)
        g3s.append(g_k.reshape(nf, B, D_F))

    out_t = _assemble(xc_t, xb_t, wb, t0_t, dt_t, g3s)
    return jnp.transpose(out_t, (2, 0, 1))


# speedup vs baseline: 1.5211x; 1.5211x over previous
import functools

import jax
import jax.numpy as jnp
from jax import lax
from jax.experimental import pallas as pl
from jax.experimental.pallas import tpu as pltpu
from jax.experimental.pallas import tpu_sc as plsc

B = 16384
N_CONT = 13
N_BINARY = 26
N_CAT = 26
VOCAB = 100000
D_F = 32
N_FEAT = N_CONT + N_BINARY + N_CAT

NC, NS = 2, 16
NW = NC * NS
TOTAL_IDX = B * N_CAT
IDX_PER_W = TOTAL_IDX // NW
CHUNK = 832
ROW = 128

BB = 1024


def _gather_body(table_hbm, idx_hbm, out_hbm, idx_v, rows_v, sem):
    wid = lax.axis_index("s") * NC + lax.axis_index("c")
    base = wid * IDX_PER_W

    @pl.loop(0, IDX_PER_W, step=CHUNK)
    def _(off):
        pltpu.sync_copy(idx_hbm.at[pl.ds(base + off, CHUNK)], idx_v)
        pltpu.async_copy(table_hbm.at[idx_v], rows_v, sem).wait()
        pltpu.sync_copy(rows_v, out_hbm.at[pl.ds(base + off, CHUNK)])


def _cat_gather(table_flat, idx_flat):
    mesh = plsc.VectorSubcoreMesh(core_axis_name="c", subcore_axis_name="s")
    k = pl.kernel(
        _gather_body,
        out_type=jax.ShapeDtypeStruct((TOTAL_IDX, ROW), jnp.float32),
        mesh=mesh,
        scratch_types=[
            pltpu.VMEM((CHUNK,), jnp.int32),
            pltpu.VMEM((CHUNK, ROW), jnp.float32),
            pltpu.SemaphoreType.DMA,
        ],
        compiler_params=pltpu.CompilerParams(use_tc_tiling_on_sc=False),
    )
    return k(table_flat, idx_flat)


def _asm_body(xc_ref, xb_ref, wb_ref, t0_ref, dt_ref, g_ref, o_ref):
    wcol = wb_ref[:, 0:1]
    bcol = wb_ref[:, 1:2]
    for i in range(N_CONT):
        o_ref[i] = wcol * xc_ref[i:i + 1, :] + bcol
    for i in range(N_BINARY):
        o_ref[N_CONT + i] = (
            t0_ref[:, i:i + 1] + dt_ref[:, i:i + 1] * xb_ref[i:i + 1, :]
        )
    for i in range(N_CAT):
        o_ref[N_CONT + N_BINARY + i] = jnp.transpose(g_ref[i][:, :D_F])


def _assemble(xc_t, xb_t, wb, t0_t, dt_t, g3):
    return pl.pallas_call(
        _asm_body,
        grid=(B // BB,),
        in_specs=[
            pl.BlockSpec((N_CONT, BB), lambda j: (0, j)),
            pl.BlockSpec((N_BINARY, BB), lambda j: (0, j)),
            pl.BlockSpec((D_F, 2), lambda j: (0, 0)),
            pl.BlockSpec((D_F, N_BINARY), lambda j: (0, 0)),
            pl.BlockSpec((D_F, N_BINARY), lambda j: (0, 0)),
            pl.BlockSpec((N_CAT, BB, ROW), lambda j: (0, j, 0)),
        ],
        out_specs=pl.BlockSpec((N_FEAT, D_F, BB), lambda j: (0, 0, j)),
        out_shape=jax.ShapeDtypeStruct((N_FEAT, D_F, B), jnp.float32),
    )(xc_t, xb_t, wb, t0_t, dt_t, g3)


def kernel(x_cont, x_binary, x_cat, W_cont, b_cont, binary_tables, cat_tables):
    xct = x_cat.T.astype(jnp.int32)
    idx = (
        xct + (jnp.arange(N_CAT, dtype=jnp.int32) * VOCAB)[:, None]
    ).reshape(TOTAL_IDX)
    table128 = jnp.pad(
        cat_tables, ((0, 0), (0, 0), (0, ROW - D_F))
    ).reshape(N_CAT * VOCAB, ROW)

    xc_t = x_cont.T
    xb_t = x_binary.T.astype(jnp.float32)
    wb = jnp.stack([W_cont[0], b_cont], axis=1)
    t0_t = binary_tables[:, 0, :].T
    dt_t = (binary_tables[:, 1, :] - binary_tables[:, 0, :]).T

    g = _cat_gather(table128, idx)
    g3 = g.reshape(N_CAT, B, ROW)

    out_t = _assemble(xc_t, xb_t, wb, t0_t, dt_t, g3)
    return jnp.transpose(out_t, (2, 0, 1))


# speedup vs baseline: 2.4261x; 1.5949x over previous
import functools

import jax
import jax.numpy as jnp
from jax import lax
from jax.experimental import pallas as pl
from jax.experimental.pallas import tpu as pltpu
from jax.experimental.pallas import tpu_sc as plsc

B = 16384
N_CONT = 13
N_BINARY = 26
N_CAT = 26
VOCAB = 100000
D_F = 32
N_FEAT = N_CONT + N_BINARY + N_CAT

NC, NS = 2, 16
NW = NC * NS
TOTAL_IDX = B * N_CAT
IDX_PER_W = TOTAL_IDX // NW
CHUNK = 832
ROW = 128

BB = 1024


VCHUNK = 12500


def _tblt_body(t_ref, o_ref, tb0, tb1, sem0, sem1):
    i = pl.program_id(0)
    tbufs = (tb0, tb1)
    sems = (sem0, sem1)
    for c in range(VOCAB // VCHUNK):
        tb, sem = tbufs[c % 2], sems[c % 2]
        if c >= 2:
            pltpu.make_async_copy(
                tb, o_ref.at[pl.ds(0, VCHUNK), :], sem
            ).wait()
        tb[:, :D_F] = jnp.transpose(
            t_ref[0, :, c * VCHUNK:(c + 1) * VCHUNK]
        )
        pltpu.make_async_copy(
            tb,
            o_ref.at[pl.ds(i * VOCAB + c * VCHUNK, VCHUNK), :],
            sem,
        ).start()
    for c in range(2):
        pltpu.make_async_copy(
            tbufs[c], o_ref.at[pl.ds(0, VCHUNK), :], sems[c]
        ).wait()


def _table_pad(cat_t):
    return pl.pallas_call(
        _tblt_body,
        grid=(N_CAT,),
        in_specs=[pl.BlockSpec((1, D_F, VOCAB), lambda i: (i, 0, 0))],
        out_specs=pl.BlockSpec(memory_space=pl.ANY),
        out_shape=jax.ShapeDtypeStruct((N_CAT * VOCAB, ROW), jnp.float32),
        scratch_shapes=[
            pltpu.VMEM((VCHUNK, ROW), jnp.float32),
            pltpu.VMEM((VCHUNK, ROW), jnp.float32),
            pltpu.SemaphoreType.DMA,
            pltpu.SemaphoreType.DMA,
        ],
    )(cat_t)


def _gather_body(table_hbm, idx_hbm, out_hbm, idx_v, rows_v, sem):
    wid = lax.axis_index("s") * NC + lax.axis_index("c")
    base = wid * IDX_PER_W

    @pl.loop(0, IDX_PER_W, step=CHUNK)
    def _(off):
        pltpu.sync_copy(idx_hbm.at[pl.ds(base + off, CHUNK)], idx_v)
        pltpu.async_copy(table_hbm.at[idx_v], rows_v, sem).wait()
        pltpu.sync_copy(rows_v, out_hbm.at[pl.ds(base + off, CHUNK)])


def _cat_gather(table_flat, idx_flat):
    mesh = plsc.VectorSubcoreMesh(core_axis_name="c", subcore_axis_name="s")
    k = pl.kernel(
        _gather_body,
        out_type=jax.ShapeDtypeStruct((TOTAL_IDX, ROW), jnp.float32),
        mesh=mesh,
        scratch_types=[
            pltpu.VMEM((CHUNK,), jnp.int32),
            pltpu.VMEM((CHUNK, ROW), jnp.float32),
            pltpu.SemaphoreType.DMA,
        ],
        compiler_params=pltpu.CompilerParams(use_tc_tiling_on_sc=False),
    )
    return k(table_flat, idx_flat)


def _asm_body(xc_ref, xb_ref, wb_ref, t0_ref, dt_ref, g_ref, o_ref):
    wcol = wb_ref[:, 0:1]
    bcol = wb_ref[:, 1:2]
    for i in range(N_CONT):
        o_ref[i] = wcol * xc_ref[i:i + 1, :] + bcol
    for i in range(N_BINARY):
        o_ref[N_CONT + i] = (
            t0_ref[:, i:i + 1] + dt_ref[:, i:i + 1] * xb_ref[i:i + 1, :]
        )
    for i in range(N_CAT):
        o_ref[N_CONT + N_BINARY + i] = jnp.transpose(g_ref[i][:, :D_F])


def _assemble(xc_t, xb_t, wb, t0_t, dt_t, g3):
    return pl.pallas_call(
        _asm_body,
        grid=(B // BB,),
        in_specs=[
            pl.BlockSpec((N_CONT, BB), lambda j: (0, j)),
            pl.BlockSpec((N_BINARY, BB), lambda j: (0, j)),
            pl.BlockSpec((D_F, 2), lambda j: (0, 0)),
            pl.BlockSpec((D_F, N_BINARY), lambda j: (0, 0)),
            pl.BlockSpec((D_F, N_BINARY), lambda j: (0, 0)),
            pl.BlockSpec((N_CAT, BB, ROW), lambda j: (0, j, 0)),
        ],
        out_specs=pl.BlockSpec((N_FEAT, D_F, BB), lambda j: (0, 0, j)),
        out_shape=jax.ShapeDtypeStruct((N_FEAT, D_F, B), jnp.float32),
    )(xc_t, xb_t, wb, t0_t, dt_t, g3)


def kernel(x_cont, x_binary, x_cat, W_cont, b_cont, binary_tables, cat_tables):
    xct = x_cat.T.astype(jnp.int32)
    idx = (
        xct + (jnp.arange(N_CAT, dtype=jnp.int32) * VOCAB)[:, None]
    ).reshape(TOTAL_IDX)
    cat_t = jnp.transpose(cat_tables, (0, 2, 1))
    table128 = _table_pad(cat_t)

    xc_t = x_cont.T
    xb_t = x_binary.T.astype(jnp.float32)
    wb = jnp.stack([W_cont[0], b_cont], axis=1)
    t0_t = binary_tables[:, 0, :].T
    dt_t = (binary_tables[:, 1, :] - binary_tables[:, 0, :]).T

    g = _cat_gather(table128, idx)
    g3 = g.reshape(N_CAT, B, ROW)

    out_t = _assemble(xc_t, xb_t, wb, t0_t, dt_t, g3)
    return jnp.transpose(out_t, (2, 0, 1))


# speedup vs baseline: 2.4397x; 1.0056x over previous
import functools

import jax
import jax.numpy as jnp
from jax import lax
from jax.experimental import pallas as pl
from jax.experimental.pallas import tpu as pltpu
from jax.experimental.pallas import tpu_sc as plsc

B = 16384
N_CONT = 13
N_BINARY = 26
N_CAT = 26
VOCAB = 100000
D_F = 32
N_FEAT = N_CONT + N_BINARY + N_CAT

NC, NS = 2, 16
NW = NC * NS
TOTAL_IDX = B * N_CAT
IDX_PER_W = TOTAL_IDX // NW
CHUNK = 832
ROW = 128

BB = 1024


VCHUNK = 12500


def _tblt_body(t_ref, o_ref, tb0, tb1, sem0, sem1):
    i = pl.program_id(0)
    tbufs = (tb0, tb1)
    sems = (sem0, sem1)
    for c in range(VOCAB // VCHUNK):
        tb, sem = tbufs[c % 2], sems[c % 2]
        if c >= 2:
            pltpu.make_async_copy(
                tb, o_ref.at[pl.ds(0, VCHUNK), :], sem
            ).wait()
        tb[:, :D_F] = jnp.transpose(
            t_ref[0, :, c * VCHUNK:(c + 1) * VCHUNK]
        )
        pltpu.make_async_copy(
            tb,
            o_ref.at[pl.ds(i * VOCAB + c * VCHUNK, VCHUNK), :],
            sem,
        ).start()
    for c in range(2):
        pltpu.make_async_copy(
            tbufs[c], o_ref.at[pl.ds(0, VCHUNK), :], sems[c]
        ).wait()


def _table_pad(cat_t, a, nf):
    return pl.pallas_call(
        _tblt_body,
        grid=(nf,),
        in_specs=[pl.BlockSpec((1, D_F, VOCAB), lambda i, a=a: (a + i, 0, 0))],
        out_specs=pl.BlockSpec(memory_space=pl.ANY),
        out_shape=jax.ShapeDtypeStruct((nf * VOCAB, ROW), jnp.float32),
        scratch_shapes=[
            pltpu.VMEM((VCHUNK, ROW), jnp.float32),
            pltpu.VMEM((VCHUNK, ROW), jnp.float32),
            pltpu.SemaphoreType.DMA,
            pltpu.SemaphoreType.DMA,
        ],
    )(cat_t)


def _mk_gather_body(n_per_w, chunk):
    def _gather_body(table_hbm, idx_hbm, out_hbm, idx_v, rows_v, sem):
        wid = lax.axis_index("s") * NC + lax.axis_index("c")
        base = wid * n_per_w

        @pl.loop(0, n_per_w, step=chunk)
        def _(off):
            pltpu.sync_copy(idx_hbm.at[pl.ds(base + off, chunk)], idx_v)
            pltpu.async_copy(table_hbm.at[idx_v], rows_v, sem).wait()
            pltpu.sync_copy(rows_v, out_hbm.at[pl.ds(base + off, chunk)])

    return _gather_body


def _cat_gather(table_flat, idx_flat, nf):
    n_idx = B * nf
    n_per_w = n_idx // NW
    chunk = n_per_w // 4
    mesh = plsc.VectorSubcoreMesh(core_axis_name="c", subcore_axis_name="s")
    k = pl.kernel(
        _mk_gather_body(n_per_w, chunk),
        out_type=jax.ShapeDtypeStruct((n_idx, ROW), jnp.float32),
        mesh=mesh,
        scratch_types=[
            pltpu.VMEM((chunk,), jnp.int32),
            pltpu.VMEM((chunk, ROW), jnp.float32),
            pltpu.SemaphoreType.DMA,
        ],
        compiler_params=pltpu.CompilerParams(use_tc_tiling_on_sc=False),
    )
    return k(table_flat, idx_flat)


FSPLITS = ((0, 7), (7, 14), (14, 20), (20, 26))


def _asm_body(xc_ref, xb_ref, wb_ref, t0_ref, dt_ref, *rest):
    g_refs, o_ref = rest[:N_CAT], rest[N_CAT]
    wcol = wb_ref[:, 0:1]
    bcol = wb_ref[:, 1:2]
    for i in range(N_CONT):
        o_ref[i] = wcol * xc_ref[i:i + 1, :] + bcol
    for i in range(N_BINARY):
        o_ref[N_CONT + i] = (
            t0_ref[:, i:i + 1] + dt_ref[:, i:i + 1] * xb_ref[i:i + 1, :]
        )
    for i in range(N_CAT):
        o_ref[N_CONT + N_BINARY + i] = jnp.transpose(g_refs[i][0][:, :D_F])


def _assemble(xc_t, xb_t, wb, t0_t, dt_t, g3s):
    in_specs = [
        pl.BlockSpec((N_CONT, BB), lambda j: (0, j)),
        pl.BlockSpec((N_BINARY, BB), lambda j: (0, j)),
        pl.BlockSpec((D_F, 2), lambda j: (0, 0)),
        pl.BlockSpec((D_F, N_BINARY), lambda j: (0, 0)),
        pl.BlockSpec((D_F, N_BINARY), lambda j: (0, 0)),
    ]
    g_args = []
    for k, (a, b) in enumerate(FSPLITS):
        for i in range(a, b):
            in_specs.append(
                pl.BlockSpec((1, BB, ROW), lambda j, li=i - a: (li, j, 0))
            )
            g_args.append(g3s[k])

    return pl.pallas_call(
        _asm_body,
        grid=(B // BB,),
        in_specs=in_specs,
        out_specs=pl.BlockSpec((N_FEAT, D_F, BB), lambda j: (0, 0, j)),
        out_shape=jax.ShapeDtypeStruct((N_FEAT, D_F, B), jnp.float32),
    )(xc_t, xb_t, wb, t0_t, dt_t, *g_args)


def kernel(x_cont, x_binary, x_cat, W_cont, b_cont, binary_tables, cat_tables):
    xct = x_cat.T.astype(jnp.int32)
    cat_t = jnp.transpose(cat_tables, (0, 2, 1))

    xc_t = x_cont.T
    xb_t = x_binary.T.astype(jnp.float32)
    wb = jnp.stack([W_cont[0], b_cont], axis=1)
    t0_t = binary_tables[:, 0, :].T
    dt_t = (binary_tables[:, 1, :] - binary_tables[:, 0, :]).T

    g3s = []
    for (a, b) in FSPLITS:
        nf = b - a
        idx_k = (
            xct[a:b] + (jnp.arange(nf, dtype=jnp.int32) * VOCAB)[:, None]
        ).reshape(B * nf)
        tbl_k = _table_pad(cat_t, a, nf)
        g_k = _cat_gather(tbl_k, idx_k, nf)
        g3s.append(g_k.reshape(nf, B, ROW))

    out_t = _assemble(xc_t, xb_t, wb, t0_t, dt_t, g3s)
    return jnp.transpose(out_t, (2, 0, 1))
